# baseline (speedup 1.0000x reference)
import functools

import jax
import jax.numpy as jnp
from jax import lax
from jax.experimental import pallas as pl
from jax.experimental.pallas import tpu as pltpu
from jax.experimental.pallas import tpu_sc as plsc

NCLS = 100000
D = 64
B = 16384
MOM = 0.9
UPD = 1.0 - MOM

NC = 2
NS = 16
NW = NC * NS
NR = 2 * NW
CR = 1568
CHUNK = 2048
NCH = B // CHUNK
BLK = 32

_Z16F = functools.partial(jnp.zeros, (16,), jnp.float32)


def _body(feat, lab, proto, out, labch, sums, counts, mi, mc, rows,
          pb, semf, semb):
    wid = lax.axis_index("s") * NC + lax.axis_index("c")
    iota = lax.iota(jnp.int32, 16)

    def range_body(rr, carry0):
        lo = (wid * 2 + rr) * CR
        hi = jnp.minimum(lo + CR, NCLS)

        def zs(v, carry):
            sums[pl.ds(v * 16, 16)] = _Z16F()
            return carry
        lax.fori_loop(0, CR * D // 16, zs, 0)

        def zc(v, carry):
            counts[pl.ds(v * 16, 16)] = _Z16F()
            return carry
        lax.fori_loop(0, (CR + 32) // 16, zc, 0)

        def chunk_body(ch, carry):
            base = ch * CHUNK
            pltpu.sync_copy(lab.at[pl.ds(base, CHUNK)], labch)

            def cmp_body(v, n):
                lv = labch[pl.ds(v * 16, 16)]
                m = (lv >= lo) & (lv < hi)
                c = plsc.cumsum(jnp.where(m, 1, 0))
                pos = jnp.where(m, n + c - 1, CHUNK + iota)
                plsc.store_scatter(mi, [pos], base + v * 16 + iota)
                plsc.store_scatter(mc, [pos], lv)
                return n + c[15]
            n = lax.fori_loop(0, CHUNK // 16, cmp_body, 0)
            mi[pl.ds(n, 16)] = wid * 16 + iota

            def grp(g, carry2):
                idxv = mi[pl.ds(g * 16, 16)]
                pltpu.async_copy(feat.at[idxv], rows, semf).wait()
                cl_vec = mc[pl.ds(g * 16, 16)] - lo
                for k in range(16):
                    @pl.when(g * 16 + k < n)
                    def _do(k=k):
                        cl = cl_vec[k]
                        off = cl * D
                        for q in range(4):
                            s = sums[pl.ds(off + q * 16, 16)]
                            sums[pl.ds(off + q * 16, 16)] = (
                                s + rows[k, pl.ds(q * 16, 16)])
                        cv = counts[pl.ds(cl, 16)]
                        counts[pl.ds(cl, 16)] = jnp.where(
                            iota == 0, cv + 1.0, cv)
                return carry2
            lax.fori_loop(0, (n + 15) // 16, grp, 0)
            return carry
        lax.fori_loop(0, NCH, chunk_body, 0)

        def blk_body(b, carry):
            rbase = b * BLK
            pltpu.sync_copy(proto.at[pl.ds(lo + rbase, BLK)], pb)
            for sb in range(BLK // 16):
                cv = counts[pl.ds(rbase + sb * 16, 16)]
                anyp = jnp.sum(jnp.where(cv > 0.0, 1, 0))

                @pl.when(anyp > 0)
                def _bl(sb=sb, cv=cv):
                    fac_v = jnp.where(cv > 0.0, UPD / cv, 0.0)
                    for k in range(16):
                        @pl.when(cv[k] > 0.0)
                        def _row(sb=sb, k=k, fac_v=fac_v):
                            off = (rbase + sb * 16 + k) * D
                            f = fac_v[k]
                            for q in range(4):
                                p = pb[sb * 16 + k, pl.ds(q * 16, 16)]
                                s = sums[pl.ds(off + q * 16, 16)]
                                pb[sb * 16 + k, pl.ds(q * 16, 16)] = (
                                    MOM * p + s * f)
            pltpu.sync_copy(pb, out.at[pl.ds(lo + rbase, BLK)])
            return carry
        lax.fori_loop(0, (hi - lo) // BLK, blk_body, 0)
        return carry0

    lax.fori_loop(0, 2, range_body, 0)


_proto_update = functools.partial(
    pl.kernel,
    out_type=jax.ShapeDtypeStruct((NCLS, D), jnp.float32),
    mesh=plsc.VectorSubcoreMesh(core_axis_name="c", subcore_axis_name="s",
                                num_cores=NC, num_subcores=NS),
    compiler_params=pltpu.CompilerParams(needs_layout_passes=False),
    scratch_types=[
        pltpu.VMEM((CHUNK,), jnp.int32),
        pltpu.VMEM((CR * D,), jnp.float32),
        pltpu.VMEM((CR + 48,), jnp.float32),
        pltpu.VMEM((CHUNK + 32,), jnp.int32),
        pltpu.VMEM((CHUNK + 32,), jnp.int32),
        pltpu.VMEM((16, 2 * D), jnp.float32),
        pltpu.VMEM((BLK, D), jnp.float32),
        pltpu.SemaphoreType.DMA,
        pltpu.SemaphoreType.DMA,
    ],
)(_body)


def kernel(features, labels, prototypes):
    feat128 = jnp.pad(features, ((0, 0), (0, 128 - D)))
    return _proto_update(feat128, labels.astype(jnp.int32), prototypes)


# speedup vs baseline: 1.2096x; 1.2096x over previous
import functools

import jax
import jax.numpy as jnp
from jax import lax
from jax.experimental import pallas as pl
from jax.experimental.pallas import tpu as pltpu
from jax.experimental.pallas import tpu_sc as plsc

NCLS = 100000
D = 64
B = 16384
MOM = 0.9
UPD = 1.0 - MOM

NC = 2
NS = 16
NW = NC * NS
NRPT = 3
NR = NRPT * NW
CR = 1056
CHUNK = 2048
NCH = B // CHUNK
BLK = 64

_Z16F = functools.partial(jnp.zeros, (16,), jnp.float32)


def _body(feat, lab, proto, out, labch, sums, counts, mi, mc,
          rowsA, rowsB, pbA, pbB, semfA, semfB, semiA, semiB, semoA, semoB):
    wid = lax.axis_index("s") * NC + lax.axis_index("c")
    iota = lax.iota(jnp.int32, 16)

    def range_body(rr, carry0):
        lo = (wid * NRPT + rr) * CR
        hi = jnp.minimum(lo + CR, NCLS)
        size = hi - lo

        @pl.when(size > 0)
        def _range():
            def zc(v, carry):
                counts[pl.ds(v * 16, 16)] = _Z16F()
                return carry
            lax.fori_loop(0, (CR + 64) // 16, zc, 0)

            def accum_group(g, n, rows):
                cl_vec = mc[pl.ds(g * 16, 16)] - lo
                for k in range(16):
                    @pl.when(g * 16 + k < n)
                    def _do(k=k):
                        cl = cl_vec[k]
                        off = cl * D
                        cv = counts[pl.ds(cl, 16)]
                        first = cv[0] == 0.0
                        for q in range(4):
                            s = sums[pl.ds(off + q * 16, 16)]
                            v = rows[k, pl.ds(q * 16, 16)]
                            sums[pl.ds(off + q * 16, 16)] = jnp.where(
                                first, v, s + v)
                        counts[pl.ds(cl, 16)] = jnp.where(
                            iota == 0, cv + 1.0, cv)

            def chunk_body(ch, carry):
                base = ch * CHUNK
                pltpu.sync_copy(lab.at[pl.ds(base, CHUNK)], labch)

                def cmp_body(v, n):
                    lv = labch[pl.ds(v * 16, 16)]
                    m = (lv >= lo) & (lv < hi)
                    anyp = plsc.all_reduce_population_count(m)[0]

                    @pl.when(anyp > 0)
                    def _st():
                        c = plsc.cumsum(jnp.where(m, 1, 0))
                        pos = jnp.where(m, n + c - 1, CHUNK + iota)
                        plsc.store_scatter(mi, [pos], base + v * 16 + iota)
                        plsc.store_scatter(mc, [pos], lv)
                    return n + anyp
                n = lax.fori_loop(0, CHUNK // 16, cmp_body, 0)
                mi[pl.ds(n, 16)] = wid * 16 + iota
                mi[pl.ds(n + 16, 16)] = wid * 16 + iota

                ngrp = (n + 15) // 16
                gprs = (ngrp + 1) // 2

                @pl.when(ngrp > 0)
                def _groups():
                    pltpu.async_copy(feat.at[mi[pl.ds(0, 16)]], rowsA, semfA)
                    pltpu.async_copy(feat.at[mi[pl.ds(16, 16)]], rowsB, semfB)

                    def pair(bb, carry2):
                        for g, rows, semf in ((bb * 2, rowsA, semfA),
                                              (bb * 2 + 1, rowsB, semfB)):
                            pltpu.make_async_copy(
                                feat.at[mi[pl.ds(0, 16)]], rows, semf).wait()
                            accum_group(g, n, rows)

                            @pl.when(bb + 1 < gprs)
                            def _pf(g=g, rows=rows, semf=semf):
                                idxv = mi[pl.ds((g + 2) * 16, 16)]
                                pltpu.async_copy(feat.at[idxv], rows, semf)
                        return carry2
                    lax.fori_loop(0, gprs, pair, 0)
                return carry
            lax.fori_loop(0, NCH, chunk_body, 0)

            def blend(pb, b):
                rbase = jnp.minimum(b * BLK, size - BLK)

                def sub(sb, carry):
                    cv = counts[pl.ds(rbase + sb * 16, 16)]
                    anyp = jnp.sum(jnp.where(cv > 0.0, 1, 0))

                    @pl.when(anyp > 0)
                    def _bl():
                        fac_v = jnp.where(cv > 0.0, UPD / cv, 0.0)
                        for k in range(16):
                            @pl.when(cv[k] > 0.0)
                            def _row(k=k):
                                off = (rbase + sb * 16 + k) * D
                                f = fac_v[k]
                                for q in range(4):
                                    p = pb[sb * 16 + k, pl.ds(q * 16, 16)]
                                    s = sums[pl.ds(off + q * 16, 16)]
                                    pb[sb * 16 + k, pl.ds(q * 16, 16)] = (
                                        MOM * p + s * f)
                    return carry
                lax.fori_loop(0, BLK // 16, sub, 0)

            def st_of(b):
                return lo + jnp.minimum(b * BLK, size - BLK)

            nb = (size + BLK - 1) // BLK
            nprs = (nb + 1) // 2
            pltpu.async_copy(proto.at[pl.ds(st_of(0), BLK)], pbA, semiA)
            pltpu.async_copy(proto.at[pl.ds(st_of(1), BLK)], pbB, semiB)

            def bpair(bb, carry):
                for idx, pb, semi, semo in ((0, pbA, semiA, semoA),
                                            (1, pbB, semiB, semoB)):
                    b = bb * 2 + idx
                    pltpu.make_async_copy(
                        proto.at[pl.ds(lo, BLK)], pb, semi).wait()
                    blend(pb, b)
                    pltpu.async_copy(pb, out.at[pl.ds(st_of(b), BLK)], semo)

                    @pl.when(bb + 1 < nprs)
                    def _pf(b=b, pb=pb, semi=semi, semo=semo):
                        pltpu.make_async_copy(
                            pb, out.at[pl.ds(lo, BLK)], semo).wait()
                        pltpu.async_copy(
                            proto.at[pl.ds(st_of(b + 2), BLK)], pb, semi)
                return carry
            lax.fori_loop(0, nprs, bpair, 0)
            pltpu.make_async_copy(pbA, out.at[pl.ds(lo, BLK)], semoA).wait()
            pltpu.make_async_copy(pbB, out.at[pl.ds(lo, BLK)], semoB).wait()
        return carry0

    lax.fori_loop(0, NRPT, range_body, 0)


_proto_update = functools.partial(
    pl.kernel,
    out_type=jax.ShapeDtypeStruct((NCLS, D), jnp.float32),
    mesh=plsc.VectorSubcoreMesh(core_axis_name="c", subcore_axis_name="s",
                                num_cores=NC, num_subcores=NS),
    compiler_params=pltpu.CompilerParams(needs_layout_passes=False),
    scratch_types=[
        pltpu.VMEM((CHUNK,), jnp.int32),
        pltpu.VMEM((CR * D,), jnp.float32),
        pltpu.VMEM((CR + 80,), jnp.float32),
        pltpu.VMEM((CHUNK + 48,), jnp.int32),
        pltpu.VMEM((CHUNK + 32,), jnp.int32),
        pltpu.VMEM((16, 2 * D), jnp.float32),
        pltpu.VMEM((16, 2 * D), jnp.float32),
        pltpu.VMEM((BLK, D), jnp.float32),
        pltpu.VMEM((BLK, D), jnp.float32),
        pltpu.SemaphoreType.DMA,
        pltpu.SemaphoreType.DMA,
        pltpu.SemaphoreType.DMA,
        pltpu.SemaphoreType.DMA,
        pltpu.SemaphoreType.DMA,
        pltpu.SemaphoreType.DMA,
    ],
)(_body)


def kernel(features, labels, prototypes):
    feat128 = jnp.pad(features, ((0, 0), (0, 128 - D)))
    return _proto_update(feat128, labels.astype(jnp.int32), prototypes)


# speedup vs baseline: 1.2933x; 1.0692x over previous
import functools

import jax
import jax.numpy as jnp
from jax import lax
from jax.experimental import pallas as pl
from jax.experimental.pallas import tpu as pltpu
from jax.experimental.pallas import tpu_sc as plsc

NCLS = 100000
D = 64
B = 16384
MOM = 0.9
UPD = 1.0 - MOM

NC = 2
NS = 16
NW = NC * NS
NRPT = 3
NR = NRPT * NW
CR = 1056
CHUNK = 2048
NCH = B // CHUNK
BLK = 48

_Z16F = functools.partial(jnp.zeros, (16,), jnp.float32)


def _body(feat, lab, proto, out, labch, sums, counts, mi, mc,
          rowsA, rowsB, pb0, pb1, pb2, semfA, semfB,
          semi0, semi1, semi2, semo0, semo1, semo2):
    wid = lax.axis_index("s") * NC + lax.axis_index("c")
    iota = lax.iota(jnp.int32, 16)

    def range_body(rr, carry0):
        lo = (wid * NRPT + rr) * CR
        hi = jnp.minimum(lo + CR, NCLS)
        size = hi - lo

        @pl.when(size > 0)
        def _range():
            def zc(v, carry):
                counts[pl.ds(v * 16, 16)] = _Z16F()
                return carry
            lax.fori_loop(0, (CR + 64) // 16, zc, 0)

            def st_of(b):
                return lo + jnp.minimum(b * BLK, size - BLK)

            nb = (size + BLK - 1) // BLK
            pltpu.async_copy(proto.at[pl.ds(st_of(0), BLK)], pb0, semi0)
            pltpu.async_copy(proto.at[pl.ds(st_of(1), BLK)], pb1, semi1)

            def accum_group(g, n, rows):
                cl_vec = mc[pl.ds(g * 16, 16)] - lo
                for k in range(16):
                    @pl.when(g * 16 + k < n)
                    def _do(k=k):
                        cl = cl_vec[k]
                        off = cl * D
                        cv = counts[pl.ds(cl, 16)]
                        first = cv[0] == 0.0
                        for q in range(4):
                            s = sums[pl.ds(off + q * 16, 16)]
                            v = rows[k, pl.ds(q * 16, 16)]
                            sums[pl.ds(off + q * 16, 16)] = jnp.where(
                                first, v, s + v)
                        counts[pl.ds(cl, 16)] = jnp.where(
                            iota == 0, cv + 1.0, cv)

            def chunk_body(ch, carry):
                base = ch * CHUNK
                pltpu.sync_copy(lab.at[pl.ds(base, CHUNK)], labch)

                def cmp_body(v, n):
                    lv = labch[pl.ds(v * 16, 16)]
                    m = (lv >= lo) & (lv < hi)
                    anyp = plsc.all_reduce_population_count(m)[0]

                    @pl.when(anyp > 0)
                    def _st():
                        c = plsc.cumsum(jnp.where(m, 1, 0))
                        pos = jnp.where(m, n + c - 1, CHUNK + iota)
                        plsc.store_scatter(mi, [pos], base + v * 16 + iota)
                        plsc.store_scatter(mc, [pos], lv)
                    return n + anyp
                n = lax.fori_loop(0, CHUNK // 16, cmp_body, 0)
                mi[pl.ds(n, 16)] = wid * 16 + iota
                mi[pl.ds(n + 16, 16)] = wid * 16 + iota

                ngrp = (n + 15) // 16
                gprs = (ngrp + 1) // 2

                @pl.when(ngrp > 0)
                def _groups():
                    pltpu.async_copy(feat.at[mi[pl.ds(0, 16)]], rowsA, semfA)
                    pltpu.async_copy(feat.at[mi[pl.ds(16, 16)]], rowsB, semfB)

                    def pair(bb, carry2):
                        for g, rows, semf in ((bb * 2, rowsA, semfA),
                                              (bb * 2 + 1, rowsB, semfB)):
                            pltpu.make_async_copy(
                                feat.at[mi[pl.ds(0, 16)]], rows, semf).wait()
                            accum_group(g, n, rows)

                            @pl.when(bb + 1 < gprs)
                            def _pf(g=g, rows=rows, semf=semf):
                                idxv = mi[pl.ds((g + 2) * 16, 16)]
                                pltpu.async_copy(feat.at[idxv], rows, semf)
                        return carry2
                    lax.fori_loop(0, gprs, pair, 0)
                return carry
            lax.fori_loop(0, NCH, chunk_body, 0)

            def blend(pb, b):
                rbase = jnp.minimum(b * BLK, size - BLK)

                def sub(sb, carry):
                    cv = counts[pl.ds(rbase + sb * 16, 16)]
                    anyp = jnp.sum(jnp.where(cv > 0.0, 1, 0))

                    @pl.when(anyp > 0)
                    def _bl():
                        fac_v = jnp.where(cv > 0.0, UPD / cv, 0.0)
                        for k in range(16):
                            @pl.when(cv[k] > 0.0)
                            def _row(k=k):
                                off = (rbase + sb * 16 + k) * D
                                f = fac_v[k]
                                for q in range(4):
                                    p = pb[sb * 16 + k, pl.ds(q * 16, 16)]
                                    s = sums[pl.ds(off + q * 16, 16)]
                                    pb[sb * 16 + k, pl.ds(q * 16, 16)] = (
                                        MOM * p + s * f)
                    return carry
                lax.fori_loop(0, BLK // 16, sub, 0)

            pbs = ((pb0, semi0, semo0), (pb1, semi1, semo1),
                   (pb2, semi2, semo2))

            def btrip(cc, carry):
                for j in range(3):
                    c = cc * 3 + j
                    pb, semi, semo = pbs[j]
                    nxt, nsemi, nsemo = pbs[(j + 2) % 3]

                    @pl.when(c < nb)
                    def _sec(c=c, pb=pb, semi=semi, semo=semo,
                             nxt=nxt, nsemi=nsemi, nsemo=nsemo):
                        pltpu.make_async_copy(
                            proto.at[pl.ds(lo, BLK)], pb, semi).wait()
                        blend(pb, c)
                        pltpu.async_copy(
                            pb, out.at[pl.ds(st_of(c), BLK)], semo)

                        @pl.when(c + 2 < nb)
                        def _pf():
                            @pl.when(c >= 1)
                            def _wo():
                                pltpu.make_async_copy(
                                    nxt, out.at[pl.ds(lo, BLK)],
                                    nsemo).wait()
                            pltpu.async_copy(
                                proto.at[pl.ds(st_of(c + 2), BLK)],
                                nxt, nsemi)
                return carry
            lax.fori_loop(0, (nb + 2) // 3, btrip, 0)
            pltpu.make_async_copy(pb0, out.at[pl.ds(lo, BLK)], semo0).wait()
            pltpu.make_async_copy(pb1, out.at[pl.ds(lo, BLK)], semo1).wait()
            pltpu.make_async_copy(pb2, out.at[pl.ds(lo, BLK)], semo2).wait()
        return carry0

    lax.fori_loop(0, NRPT, range_body, 0)


_proto_update = functools.partial(
    pl.kernel,
    out_type=jax.ShapeDtypeStruct((NCLS, D), jnp.float32),
    mesh=plsc.VectorSubcoreMesh(core_axis_name="c", subcore_axis_name="s",
                                num_cores=NC, num_subcores=NS),
    compiler_params=pltpu.CompilerParams(needs_layout_passes=False),
    scratch_types=[
        pltpu.VMEM((CHUNK,), jnp.int32),
        pltpu.VMEM((CR * D,), jnp.float32),
        pltpu.VMEM((CR + 80,), jnp.float32),
        pltpu.VMEM((CHUNK + 48,), jnp.int32),
        pltpu.VMEM((CHUNK + 32,), jnp.int32),
        pltpu.VMEM((16, 2 * D), jnp.float32),
        pltpu.VMEM((16, 2 * D), jnp.float32),
        pltpu.VMEM((BLK, D), jnp.float32),
        pltpu.VMEM((BLK, D), jnp.float32),
        pltpu.VMEM((BLK, D), jnp.float32),
        pltpu.SemaphoreType.DMA,
        pltpu.SemaphoreType.DMA,
        pltpu.SemaphoreType.DMA,
        pltpu.SemaphoreType.DMA,
        pltpu.SemaphoreType.DMA,
        pltpu.SemaphoreType.DMA,
        pltpu.SemaphoreType.DMA,
        pltpu.SemaphoreType.DMA,
    ],
)(_body)


def kernel(features, labels, prototypes):
    feat128 = jnp.pad(features, ((0, 0), (0, 128 - D)))
    return _proto_update(feat128, labels.astype(jnp.int32), prototypes)
